# Initial kernel scaffold; baseline (speedup 1.0000x reference)
import jax
import jax.numpy as jnp
from jax.experimental import pallas as pl


def kernel(node_features, node_attrs, edge_attrs, edge_embedding, edge_index, W_lin1, W_fc1, W_fc2, W_lin2, W_sc):
    raise NotImplementedError("write your pallas kernel here")


# speedup vs baseline: 1.5261x; 1.5261x over previous
import functools

import numpy as np
import jax
import jax.numpy as jnp
from jax import lax
from jax.experimental import pallas as pl
from jax.experimental.pallas import tpu as pltpu
from jax.experimental.pallas import tpu_sc as plsc

_zz = np.linspace(-10.0, 10.0, 200001)
_pdf = np.exp(-0.5 * _zz ** 2) / np.sqrt(2.0 * np.pi)
_silu_np = _zz / (1.0 + np.exp(-_zz))
_CST = float(1.0 / np.sqrt(np.trapz(_silu_np ** 2 * _pdf, _zz)))

_N = 10000
_E = 320000
_D = 128
_A = 16
_B = 8
_H = 64

_NTILES = 32
_CHUNK = 128
_NCHUNK = 80
_EPW = _CHUNK * _NCHUNK
_EPAD = _EPW * _NTILES

_INV_D = 1.0 / np.sqrt(float(_D))
_INV_B = 1.0 / np.sqrt(float(_B))
_INV_H = 1.0 / np.sqrt(float(_H))
_INV_DA = 1.0 / np.sqrt(float(_D * _A))


def _nsilu(v):
    return v * jax.nn.sigmoid(v) * _CST


def _node_pre_body(x0_ref, attr_ref, wlin1_ref, wsc_ref, x_ref, sc_ref):
    x0 = x0_ref[...]
    at = attr_ref[...]
    x_ref[...] = lax.dot_general(
        x0, wlin1_ref[...] * _INV_D, (((1,), (0,)), ((), ())),
        preferred_element_type=jnp.float32)
    acc = jnp.zeros_like(x0)
    for v in range(_A):
        acc += lax.dot_general(
            x0 * at[:, v][:, None], wsc_ref[:, v, :], (((1,), (0,)), ((), ())),
            preferred_element_type=jnp.float32)
    sc_ref[...] = acc * _INV_DA


_NB = 1000


def _node_pre(x0, attrs, W_lin1, W_sc):
    grid = (_N // _NB,)
    return pl.pallas_call(
        _node_pre_body,
        grid=grid,
        in_specs=[
            pl.BlockSpec((_NB, _D), lambda i: (i, 0)),
            pl.BlockSpec((_NB, _A), lambda i: (i, 0)),
            pl.BlockSpec((_D, _D), lambda i: (0, 0)),
            pl.BlockSpec((_D, _A, _D), lambda i: (0, 0, 0)),
        ],
        out_specs=[
            pl.BlockSpec((_NB, _D), lambda i: (i, 0)),
            pl.BlockSpec((_NB, _D), lambda i: (i, 0)),
        ],
        out_shape=[
            jax.ShapeDtypeStruct((_N, _D), jnp.float32),
            jax.ShapeDtypeStruct((_N, _D), jnp.float32),
        ],
    )(x0, attrs, W_lin1, W_sc)


_EB = 4096


def _edge_w_body(ee_ref, ea_ref, wfc1_ref, wfc2_ref, w_ref):
    h = _nsilu(lax.dot_general(
        ee_ref[...], wfc1_ref[...] * _INV_B, (((1,), (0,)), ((), ())),
        preferred_element_type=jnp.float32))
    w = lax.dot_general(
        h, wfc2_ref[...] * _INV_H, (((1,), (0,)), ((), ())),
        preferred_element_type=jnp.float32)
    w_ref[...] = w * ea_ref[...]


def _edge_w(ee_p, ea_p, W_fc1, W_fc2):
    grid = (_EPAD // _EB,)
    return pl.pallas_call(
        _edge_w_body,
        grid=grid,
        in_specs=[
            pl.BlockSpec((_EB, _B), lambda i: (i, 0)),
            pl.BlockSpec((_EB, 1), lambda i: (i, 0)),
            pl.BlockSpec((_B, _H), lambda i: (0, 0)),
            pl.BlockSpec((_H, _D), lambda i: (0, 0)),
        ],
        out_specs=pl.BlockSpec((_EB, _D), lambda i: (i, 0)),
        out_shape=jax.ShapeDtypeStruct((_EPAD, _D), jnp.float32),
    )(ee_p, ea_p, W_fc1, W_fc2)


_SC_MESH = plsc.VectorSubcoreMesh(
    core_axis_name="c", subcore_axis_name="s", num_cores=2, num_subcores=16)


@functools.partial(
    pl.kernel,
    out_type=jax.ShapeDtypeStruct((2, _N, _D), jnp.float32),
    mesh=_SC_MESH,
    scratch_types=[
        pltpu.VMEM((_CHUNK,), jnp.int32),
        pltpu.VMEM((_CHUNK,), jnp.int32),
        pltpu.VMEM((_CHUNK, _D), jnp.float32),
        pltpu.VMEM((_CHUNK, _D), jnp.float32),
        pltpu.VMEM_SHARED((_N, _D), jnp.float32),
        pltpu.SemaphoreType.DMA,
    ],
)
def _sc_scatter(x_hbm, w_hbm, ei_hbm, ej_hbm, zeros_hbm, out_hbm,
                ei_v, ej_v, rows_v, w_v, agg_sh, sem):
    c = lax.axis_index("c")
    s = lax.axis_index("s")
    wid = s * 2 + c

    @pl.when(s == 0)
    def _zero():
        pltpu.sync_copy(zeros_hbm, agg_sh)

    plsc.subcore_barrier()

    base0 = wid * _EPW

    def chunk_body(ci, carry):
        base = base0 + ci * _CHUNK
        pltpu.sync_copy(ei_hbm.at[pl.ds(base, _CHUNK)], ei_v)
        pltpu.sync_copy(ej_hbm.at[pl.ds(base, _CHUNK)], ej_v)
        pltpu.async_copy(x_hbm.at[ei_v], rows_v, sem).wait()
        pltpu.sync_copy(w_hbm.at[pl.ds(base, _CHUNK), :], w_v)

        def mul_body(e, cc):
            for k in range(_D // 16):
                sl = pl.ds(k * 16, 16)
                rows_v[e, sl] = rows_v[e, sl] * w_v[e, sl]
            return cc

        lax.fori_loop(0, _CHUNK, mul_body, 0)
        pltpu.sync_copy(rows_v, agg_sh.at[ej_v], add=True)
        return carry

    lax.fori_loop(0, _NCHUNK, chunk_body, 0)
    plsc.subcore_barrier()

    @pl.when(s == 0)
    def _flush():
        pltpu.sync_copy(agg_sh, out_hbm.at[c])


def _post_body(parts_ref, sc_ref, x0_ref, wlin2_ref, out_ref):
    agg = parts_ref[0] + parts_ref[1]
    t = lax.dot_general(
        agg, wlin2_ref[...] * _INV_D, (((1,), (0,)), ((), ())),
        preferred_element_type=jnp.float32) + sc_ref[...]
    out_ref[...] = _nsilu(t) + x0_ref[...]


def _post(parts, sc, x0, W_lin2):
    grid = (_N // _NB,)
    return pl.pallas_call(
        _post_body,
        grid=grid,
        in_specs=[
            pl.BlockSpec((2, _NB, _D), lambda i: (0, i, 0)),
            pl.BlockSpec((_NB, _D), lambda i: (i, 0)),
            pl.BlockSpec((_NB, _D), lambda i: (i, 0)),
            pl.BlockSpec((_D, _D), lambda i: (0, 0)),
        ],
        out_specs=pl.BlockSpec((_NB, _D), lambda i: (i, 0)),
        out_shape=jax.ShapeDtypeStruct((_N, _D), jnp.float32),
    )(parts, sc, x0, W_lin2)


def kernel(node_features, node_attrs, edge_attrs, edge_embedding, edge_index,
           W_lin1, W_fc1, W_fc2, W_lin2, W_sc):
    pad = _EPAD - _E
    ei_p = jnp.concatenate([edge_index[0], jnp.zeros((pad,), jnp.int32)])
    ej_p = jnp.concatenate([edge_index[1], jnp.zeros((pad,), jnp.int32)])
    ee_p = jnp.concatenate([edge_embedding, jnp.zeros((pad, _B), jnp.float32)])
    ea_p = jnp.concatenate([edge_attrs, jnp.zeros((pad, 1), jnp.float32)])

    x, sc = _node_pre(node_features, node_attrs, W_lin1, W_sc)
    wprime = _edge_w(ee_p, ea_p, W_fc1, W_fc2)
    zeros = jnp.zeros((_N, _D), jnp.float32)
    parts = _sc_scatter(x, wprime, ei_p, ej_p, zeros)
    return _post(parts, sc, node_features, W_lin2)
